# Initial kernel scaffold; baseline (speedup 1.0000x reference)
import jax
import jax.numpy as jnp
from jax.experimental import pallas as pl


def kernel(feat, edge_index, khops, W1, b1, W2, b2, fc1_w, fc1_b, fc2_w, fc2_b):
    raise NotImplementedError("write your pallas kernel here")


# speedup vs baseline: 3.7765x; 3.7765x over previous
import functools

import jax
import jax.numpy as jnp
from jax import lax
from jax.experimental import pallas as pl
from jax.experimental.pallas import tpu as pltpu
from jax.experimental.pallas import tpu_sc as plsc

N = 4096
E = 65536
IN_DIM = 512
HID_DIM = 256
OUT_DIM = 128
TEMP = 0.5
ALPHA = 0.8

_NS = 16
_NW = 2 * _NS
_EPW = E // _NW
_CH = 128
_NCH = _EPW // _CH
_STR = N // _NS

_RBS = 256


def _sc_mesh():
    return plsc.VectorSubcoreMesh(core_axis_name="c", subcore_axis_name="s")


def _fill_zero(ref, rows, cols):
    def row(i, carry):
        def col(j, carry2):
            ref[i, pl.ds(j * 16, 16)] = jnp.zeros((16,), jnp.float32)
            return carry2
        return lax.fori_loop(0, cols // 16, col, carry)
    lax.fori_loop(0, rows, row, 0)


@functools.cache
def _make_deg_kernel():
    @functools.partial(
        pl.kernel,
        out_type=(jax.ShapeDtypeStruct((2, N, 256), jnp.float32),
                  jax.ShapeDtypeStruct((2, N, 256), jnp.float32)),
        mesh=_sc_mesh(),
        scratch_types=[
            pltpu.VMEM((_CH,), jnp.int32),
            pltpu.VMEM((_CH,), jnp.int32),
            pltpu.VMEM((_CH, 256), jnp.float32),
            pltpu.VMEM((_STR, 256), jnp.float32),
        ],
    )
    def deg(src_hbm, dst_hbm, out_src, out_dst, si_v, di_v, ones_v, zero_v):
        c = lax.axis_index("c")
        s = lax.axis_index("s")

        def fill_ones(i, carry):
            def fc(j, carry2):
                ones_v[i, pl.ds(j * 16, 16)] = jnp.full((16,), 1.0,
                                                        jnp.float32)
                return carry2
            return lax.fori_loop(0, 256 // 16, fc, carry)
        lax.fori_loop(0, _CH, fill_ones, 0)
        _fill_zero(zero_v, _STR, 256)

        pltpu.sync_copy(zero_v, out_src.at[c, pl.ds(s * _STR, _STR)])
        pltpu.sync_copy(zero_v, out_dst.at[c, pl.ds(s * _STR, _STR)])
        plsc.subcore_barrier()

        def body(i, carry):
            base = (c * _NS + s) * _EPW + i * _CH
            pltpu.sync_copy(src_hbm.at[pl.ds(base, _CH)], si_v)
            pltpu.sync_copy(dst_hbm.at[pl.ds(base, _CH)], di_v)
            pltpu.sync_copy(ones_v, out_src.at[c].at[si_v], add=True)
            pltpu.sync_copy(ones_v, out_dst.at[c].at[di_v], add=True)
            return carry
        lax.fori_loop(0, _NCH, body, 0)

    return deg


@functools.cache
def _make_segsum(D):
    zrows = (128 * 256) // D

    @functools.partial(
        pl.kernel,
        out_type=jax.ShapeDtypeStruct((2, N, D), jnp.float32),
        mesh=_sc_mesh(),
        scratch_types=[
            pltpu.VMEM((_CH,), jnp.int32),
            pltpu.VMEM((_CH,), jnp.int32),
            pltpu.VMEM((_CH, D), jnp.float32),
            pltpu.VMEM((zrows, D), jnp.float32),
            pltpu.SemaphoreType.DMA,
        ],
    )
    def seg(tab_hbm, src_hbm, dst_hbm, out_hbm, si_v, di_v, rows_v, zero_v,
            sem):
        c = lax.axis_index("c")
        s = lax.axis_index("s")

        _fill_zero(zero_v, zrows, D)

        def zcp(i, carry):
            pltpu.sync_copy(
                zero_v, out_hbm.at[c, pl.ds(s * _STR + i * zrows, zrows)])
            return carry
        lax.fori_loop(0, _STR // zrows, zcp, 0)
        plsc.subcore_barrier()

        def body(i, carry):
            base = (c * _NS + s) * _EPW + i * _CH
            pltpu.sync_copy(src_hbm.at[pl.ds(base, _CH)], si_v)
            pltpu.sync_copy(dst_hbm.at[pl.ds(base, _CH)], di_v)
            pltpu.async_copy(tab_hbm.at[si_v], rows_v, sem).wait()
            pltpu.sync_copy(rows_v, out_hbm.at[c].at[di_v], add=True)
            return carry
        lax.fori_loop(0, _NCH, body, 0)

    return seg


def _tc1_body(feat_ref, hs_ref, w_ref, out_ref):
    deg = hs_ref[0, :, 0:1] + hs_ref[1, :, 0:1]
    so = lax.rsqrt(jnp.maximum(deg, 1.0))
    out_ref[...] = jnp.dot(feat_ref[...] * so, w_ref[...],
                           preferred_element_type=jnp.float32)


_tc1 = pl.pallas_call(
    _tc1_body,
    out_shape=jax.ShapeDtypeStruct((N, 2 * HID_DIM), jnp.float32),
)


def _tc2_body(agg_ref, hs_ref, hd_ref, b1_ref, w2_ref, out_ref):
    dego = hs_ref[0, :, 0:1] + hs_ref[1, :, 0:1]
    degi = hd_ref[0, :, 0:1] + hd_ref[1, :, 0:1]
    so = lax.rsqrt(jnp.maximum(dego, 1.0))
    si = lax.rsqrt(jnp.maximum(degi, 1.0))
    h = agg_ref[0] + agg_ref[1]
    h = jnp.maximum(h * si + b1_ref[...], 0.0) * so
    out_ref[...] = jnp.dot(h, w2_ref[...], preferred_element_type=jnp.float32)


_tc2 = pl.pallas_call(
    _tc2_body,
    out_shape=jax.ShapeDtypeStruct((N, HID_DIM), jnp.float32),
)


def _tc3_body(agg_ref, hd_ref, b2_ref, fc1w_ref, fc1b_ref, fc2w_ref,
              fc2b_ref, kh_ref, out_ref, zn_s, k0_s, k1_s):
    i = pl.program_id(0)

    @pl.when(i == 0)
    def _():
        degi = hd_ref[0, :, 0:1] + hd_ref[1, :, 0:1]
        si = lax.rsqrt(jnp.maximum(degi, 1.0))
        h = agg_ref[0] + agg_ref[1]
        h = jnp.maximum(h * si + b2_ref[...], 0.0)
        p = jnp.dot(h, fc1w_ref[...], preferred_element_type=jnp.float32)
        p = p + fc1b_ref[...]
        p = jnp.where(p > 0.0, p, jnp.exp(jnp.minimum(p, 0.0)) - 1.0)
        z = jnp.dot(p, fc2w_ref[...], preferred_element_type=jnp.float32)
        z = z + fc2b_ref[...]
        nrm = jnp.sqrt(jnp.sum(z * z, axis=1, keepdims=True))
        zn_s[...] = z / jnp.maximum(nrm, 1e-12)
        k0_s[...] = jnp.zeros((1, N), jnp.float32)
        k1_s[...] = jnp.zeros((1, N), jnp.float32)

    zb = zn_s[pl.ds(i * _RBS, _RBS), :]
    sim = lax.dot_general(zb, zn_s[...], (((1,), (1,)), ((), ())),
                          preferred_element_type=jnp.float32)
    sim = jnp.exp(sim * (1.0 / TEMP))
    k0_s[...] += jnp.sum(sim * kh_ref[0], axis=0)[None, :]
    k1_s[...] += jnp.sum(sim * kh_ref[1], axis=0)[None, :]

    @pl.when(i == pl.num_programs(0) - 1)
    def _():
        k0 = k0_s[...]
        k1 = k1_s[...]
        mask = (k0 != 0.0) & (k1 != 0.0)
        denom = jnp.where(mask, k0 + k1, 1.0)
        ratio = jnp.where(mask, k0 / denom, 1.0)
        valid = mask & (ratio < ALPHA)
        loss = -jnp.sum(jnp.where(valid, jnp.log(jnp.where(valid, ratio, 1.0)),
                                  0.0), keepdims=True)
        out_ref[...] = loss


_tc3 = pl.pallas_call(
    _tc3_body,
    grid=(N // _RBS,),
    in_specs=[
        pl.BlockSpec((2, N, HID_DIM), lambda i: (0, 0, 0)),
        pl.BlockSpec((2, N, 256), lambda i: (0, 0, 0)),
        pl.BlockSpec((1, HID_DIM), lambda i: (0, 0)),
        pl.BlockSpec((HID_DIM, OUT_DIM), lambda i: (0, 0)),
        pl.BlockSpec((1, OUT_DIM), lambda i: (0, 0)),
        pl.BlockSpec((OUT_DIM, HID_DIM), lambda i: (0, 0)),
        pl.BlockSpec((1, HID_DIM), lambda i: (0, 0)),
        pl.BlockSpec((2, _RBS, N), lambda i: (0, i, 0)),
    ],
    out_specs=pl.BlockSpec((1, 1), lambda i: (0, 0)),
    out_shape=jax.ShapeDtypeStruct((1, 1), jnp.float32),
    scratch_shapes=[
        pltpu.VMEM((N, HID_DIM), jnp.float32),
        pltpu.VMEM((1, N), jnp.float32),
        pltpu.VMEM((1, N), jnp.float32),
    ],
)


def kernel(feat, edge_index, khops, W1, b1, W2, b2, fc1_w, fc1_b, fc2_w,
           fc2_b):
    src = edge_index[0]
    dst = edge_index[1]

    hist_src, hist_dst = _make_deg_kernel()(src, dst)
    t1 = _tc1(feat, hist_src, W1)
    agg1 = _make_segsum(2 * HID_DIM)(t1, src, dst)
    t2 = _tc2(agg1, hist_src, hist_dst, b1.reshape(1, -1), W2)
    agg2 = _make_segsum(HID_DIM)(t2, src, dst)
    loss = _tc3(agg2, hist_dst, b2.reshape(1, -1), fc1_w, fc1_b.reshape(1, -1),
                fc2_w, fc2_b.reshape(1, -1), khops)
    return loss[0, 0]


# speedup vs baseline: 4.2031x; 1.1130x over previous
import functools

import jax
import jax.numpy as jnp
from jax import lax
from jax.experimental import pallas as pl
from jax.experimental.pallas import tpu as pltpu
from jax.experimental.pallas import tpu_sc as plsc

N = 4096
E = 65536
IN_DIM = 512
HID_DIM = 256
OUT_DIM = 128
TEMP = 0.5
ALPHA = 0.8

_NS = 16
_NW = 2 * _NS
_EPW = E // _NW
_CH = 128
_NCH = _EPW // _CH
_STR = N // _NS

_RBS = 256


def _sc_mesh():
    return plsc.VectorSubcoreMesh(core_axis_name="c", subcore_axis_name="s")


def _fill_zero(ref, rows, cols):
    def row(i, carry):
        def col(j, carry2):
            ref[i, pl.ds(j * 16, 16)] = jnp.zeros((16,), jnp.float32)
            return carry2
        return lax.fori_loop(0, cols // 16, col, carry)
    lax.fori_loop(0, rows, row, 0)


@functools.cache
def _make_deg_kernel():
    @functools.partial(
        pl.kernel,
        out_type=(jax.ShapeDtypeStruct((2, N, 256), jnp.float32),
                  jax.ShapeDtypeStruct((2, N, 256), jnp.float32)),
        mesh=_sc_mesh(),
        scratch_types=[
            pltpu.VMEM((_NCH, _CH), jnp.int32),
            pltpu.VMEM((_NCH, _CH), jnp.int32),
            pltpu.VMEM((_CH, 256), jnp.float32),
            pltpu.VMEM((_STR, 256), jnp.float32),
            pltpu.SemaphoreType.DMA,
        ],
    )
    def deg(src_hbm, dst_hbm, out_src, out_dst, si_v, di_v, ones_v, zero_v,
            sem):
        c = lax.axis_index("c")
        s = lax.axis_index("s")
        w = c * _NS + s

        def fill_ones(i, carry):
            def fc(j, carry2):
                ones_v[i, pl.ds(j * 16, 16)] = jnp.full((16,), 1.0,
                                                        jnp.float32)
                return carry2
            return lax.fori_loop(0, 256 // 16, fc, carry)
        lax.fori_loop(0, _CH, fill_ones, 0)
        _fill_zero(zero_v, _STR, 256)

        pltpu.sync_copy(src_hbm.at[pl.ds(w * _NCH, _NCH)], si_v)
        pltpu.sync_copy(dst_hbm.at[pl.ds(w * _NCH, _NCH)], di_v)
        pltpu.sync_copy(zero_v, out_src.at[c, pl.ds(s * _STR, _STR)])
        pltpu.sync_copy(zero_v, out_dst.at[c, pl.ds(s * _STR, _STR)])
        plsc.subcore_barrier()

        handles = []
        for i in range(_NCH):
            handles.append(pltpu.async_copy(
                ones_v, out_src.at[c].at[si_v.at[i]], sem, add=True))
            handles.append(pltpu.async_copy(
                ones_v, out_dst.at[c].at[di_v.at[i]], sem, add=True))
        for h in handles:
            h.wait()

    return deg


@functools.cache
def _make_segsum(D, ch):
    nch = _EPW // ch
    zrows = (64 * 512) // D

    @functools.partial(
        pl.kernel,
        out_type=jax.ShapeDtypeStruct((2, N, D), jnp.float32),
        mesh=_sc_mesh(),
        scratch_types=[
            pltpu.VMEM((nch, ch), jnp.int32),
            pltpu.VMEM((nch, ch), jnp.int32),
            pltpu.VMEM((ch, D), jnp.float32),
            pltpu.VMEM((ch, D), jnp.float32),
            pltpu.SemaphoreType.DMA,
            pltpu.SemaphoreType.DMA,
            pltpu.SemaphoreType.DMA,
            pltpu.SemaphoreType.DMA,
        ],
    )
    def seg(tab_hbm, src_hbm, dst_hbm, out_hbm, si_v, di_v, rows0, rows1,
            gs0, gs1, ss0, ss1):
        c = lax.axis_index("c")
        s = lax.axis_index("s")
        w = c * _NS + s
        rows = (rows0, rows1)
        gsem = (gs0, gs1)
        ssem = (ss0, ss1)

        _fill_zero(rows0, zrows, D)

        pltpu.sync_copy(src_hbm.at[pl.ds(w * nch, nch)], si_v)
        pltpu.sync_copy(dst_hbm.at[pl.ds(w * nch, nch)], di_v)

        def zcp(i, carry):
            pltpu.sync_copy(
                rows0.at[pl.ds(0, zrows)],
                out_hbm.at[c, pl.ds(s * _STR + i * zrows, zrows)])
            return carry
        lax.fori_loop(0, _STR // zrows, zcp, 0)
        plsc.subcore_barrier()

        gh = [None, None]
        sh = [None, None]
        gh[0] = pltpu.async_copy(tab_hbm.at[si_v.at[0]], rows[0], gsem[0])
        for i in range(nch):
            b = i % 2
            nb = (i + 1) % 2
            if i + 1 < nch:
                if sh[nb] is not None:
                    sh[nb].wait()
                gh[nb] = pltpu.async_copy(
                    tab_hbm.at[si_v.at[i + 1]], rows[nb], gsem[nb])
            gh[b].wait()
            sh[b] = pltpu.async_copy(
                rows[b], out_hbm.at[c].at[di_v.at[i]], ssem[b], add=True)
        for b in range(2):
            if sh[b] is not None:
                sh[b].wait()

    return seg


def _tc1_body(feat_ref, hs_ref, w_ref, out_ref):
    deg = hs_ref[0, :, 0:1] + hs_ref[1, :, 0:1]
    so = lax.rsqrt(jnp.maximum(deg, 1.0))
    out_ref[...] = jnp.dot(feat_ref[...] * so, w_ref[...],
                           preferred_element_type=jnp.float32)


_tc1 = pl.pallas_call(
    _tc1_body,
    out_shape=jax.ShapeDtypeStruct((N, 2 * HID_DIM), jnp.float32),
)


def _tc2_body(agg_ref, hs_ref, hd_ref, b1_ref, w2_ref, out_ref):
    dego = hs_ref[0, :, 0:1] + hs_ref[1, :, 0:1]
    degi = hd_ref[0, :, 0:1] + hd_ref[1, :, 0:1]
    so = lax.rsqrt(jnp.maximum(dego, 1.0))
    si = lax.rsqrt(jnp.maximum(degi, 1.0))
    h = agg_ref[0] + agg_ref[1]
    h = jnp.maximum(h * si + b1_ref[...], 0.0) * so
    out_ref[...] = jnp.dot(h, w2_ref[...], preferred_element_type=jnp.float32)


_tc2 = pl.pallas_call(
    _tc2_body,
    out_shape=jax.ShapeDtypeStruct((N, HID_DIM), jnp.float32),
)


def _tc3_body(agg_ref, hd_ref, b2_ref, fc1w_ref, fc1b_ref, fc2w_ref,
              fc2b_ref, kh_ref, out_ref, zn_s, k0_s, k1_s):
    i = pl.program_id(0)

    @pl.when(i == 0)
    def _():
        degi = hd_ref[0, :, 0:1] + hd_ref[1, :, 0:1]
        si = lax.rsqrt(jnp.maximum(degi, 1.0))
        h = agg_ref[0] + agg_ref[1]
        h = jnp.maximum(h * si + b2_ref[...], 0.0)
        p = jnp.dot(h, fc1w_ref[...], preferred_element_type=jnp.float32)
        p = p + fc1b_ref[...]
        p = jnp.where(p > 0.0, p, jnp.exp(jnp.minimum(p, 0.0)) - 1.0)
        z = jnp.dot(p, fc2w_ref[...], preferred_element_type=jnp.float32)
        z = z + fc2b_ref[...]
        nrm = jnp.sqrt(jnp.sum(z * z, axis=1, keepdims=True))
        zn_s[...] = z / jnp.maximum(nrm, 1e-12)
        k0_s[...] = jnp.zeros((1, N), jnp.float32)
        k1_s[...] = jnp.zeros((1, N), jnp.float32)

    zb = zn_s[pl.ds(i * _RBS, _RBS), :]
    sim = lax.dot_general(zb, zn_s[...], (((1,), (1,)), ((), ())),
                          preferred_element_type=jnp.float32)
    sim = jnp.exp(sim * (1.0 / TEMP))
    k0_s[...] += jnp.sum(sim * kh_ref[0], axis=0)[None, :]
    k1_s[...] += jnp.sum(sim * kh_ref[1], axis=0)[None, :]

    @pl.when(i == pl.num_programs(0) - 1)
    def _():
        k0 = k0_s[...]
        k1 = k1_s[...]
        mask = (k0 != 0.0) & (k1 != 0.0)
        denom = jnp.where(mask, k0 + k1, 1.0)
        ratio = jnp.where(mask, k0 / denom, 1.0)
        valid = mask & (ratio < ALPHA)
        loss = -jnp.sum(jnp.where(valid, jnp.log(jnp.where(valid, ratio, 1.0)),
                                  0.0), keepdims=True)
        out_ref[...] = loss


_tc3 = pl.pallas_call(
    _tc3_body,
    grid=(N // _RBS,),
    in_specs=[
        pl.BlockSpec((2, N, HID_DIM), lambda i: (0, 0, 0)),
        pl.BlockSpec((2, N, 256), lambda i: (0, 0, 0)),
        pl.BlockSpec((1, HID_DIM), lambda i: (0, 0)),
        pl.BlockSpec((HID_DIM, OUT_DIM), lambda i: (0, 0)),
        pl.BlockSpec((1, OUT_DIM), lambda i: (0, 0)),
        pl.BlockSpec((OUT_DIM, HID_DIM), lambda i: (0, 0)),
        pl.BlockSpec((1, HID_DIM), lambda i: (0, 0)),
        pl.BlockSpec((2, _RBS, N), lambda i: (0, i, 0)),
    ],
    out_specs=pl.BlockSpec((1, 1), lambda i: (0, 0)),
    out_shape=jax.ShapeDtypeStruct((1, 1), jnp.float32),
    scratch_shapes=[
        pltpu.VMEM((N, HID_DIM), jnp.float32),
        pltpu.VMEM((1, N), jnp.float32),
        pltpu.VMEM((1, N), jnp.float32),
    ],
)


def kernel(feat, edge_index, khops, W1, b1, W2, b2, fc1_w, fc1_b, fc2_w,
           fc2_b):
    src = edge_index[0]
    dst = edge_index[1]
    src128 = src.reshape(E // _CH, _CH)
    dst128 = dst.reshape(E // _CH, _CH)
    src64 = src.reshape(E // 64, 64)
    dst64 = dst.reshape(E // 64, 64)

    hist_src, hist_dst = _make_deg_kernel()(src128, dst128)
    t1 = _tc1(feat, hist_src, W1)
    agg1 = _make_segsum(2 * HID_DIM, 64)(t1, src64, dst64)
    t2 = _tc2(agg1, hist_src, hist_dst, b1.reshape(1, -1), W2)
    agg2 = _make_segsum(HID_DIM, _CH)(t2, src128, dst128)
    loss = _tc3(agg2, hist_dst, b2.reshape(1, -1), fc1_w, fc1_b.reshape(1, -1),
                fc2_w, fc2_b.reshape(1, -1), khops)
    return loss[0, 0]


# speedup vs baseline: 5.0917x; 1.2114x over previous
import functools

import jax
import jax.numpy as jnp
from jax import lax
from jax.experimental import pallas as pl
from jax.experimental.pallas import tpu as pltpu
from jax.experimental.pallas import tpu_sc as plsc

N = 4096
E = 65536
IN_DIM = 512
HID_DIM = 256
OUT_DIM = 128
TEMP = 0.5
ALPHA = 0.8

_NS = 16
_NW = 2 * _NS
_EPW = E // _NW
_CH = 128
_NCH = _EPW // _CH
_STR = N // _NS

_RBS = 256


def _sc_mesh():
    return plsc.VectorSubcoreMesh(core_axis_name="c", subcore_axis_name="s")


def _fill_zero(ref, rows, cols):
    def row(i, carry):
        def col(j, carry2):
            ref[i, pl.ds(j * 16, 16)] = jnp.zeros((16,), jnp.float32)
            return carry2
        return lax.fori_loop(0, cols // 16, col, carry)
    lax.fori_loop(0, rows, row, 0)


@functools.cache
def _make_deg_kernel():
    @functools.partial(
        pl.kernel,
        out_type=jax.ShapeDtypeStruct((2, _NW, N), jnp.float32),
        mesh=_sc_mesh(),
        scratch_types=[
            pltpu.VMEM((_NCH, _CH), jnp.int32),
            pltpu.VMEM((_NCH, _CH), jnp.int32),
            pltpu.VMEM((N,), jnp.float32),
            pltpu.VMEM((N,), jnp.float32),
        ],
        compiler_params=pltpu.CompilerParams(needs_layout_passes=False),
    )
    def deg(src_hbm, dst_hbm, out, si_v, di_v, hs_v, hd_v):
        c = lax.axis_index("c")
        s = lax.axis_index("s")
        w = c * _NS + s

        def z(i, carry):
            hs_v[pl.ds(i * 16, 16)] = jnp.zeros((16,), jnp.float32)
            hd_v[pl.ds(i * 16, 16)] = jnp.zeros((16,), jnp.float32)
            return carry
        lax.fori_loop(0, N // 16, z, 0)

        pltpu.sync_copy(src_hbm.at[pl.ds(w * _NCH, _NCH)], si_v)
        pltpu.sync_copy(dst_hbm.at[pl.ds(w * _NCH, _NCH)], di_v)

        ones16 = jnp.full((16,), 1.0, jnp.float32)

        def chunk(i, carry):
            def sub(j, carry2):
                plsc.addupdate_scatter(hs_v, [si_v[i, pl.ds(j * 16, 16)]],
                                       ones16)
                plsc.addupdate_scatter(hd_v, [di_v[i, pl.ds(j * 16, 16)]],
                                       ones16)
                return carry2
            return lax.fori_loop(0, _CH // 16, sub, carry)
        lax.fori_loop(0, _NCH, chunk, 0)

        pltpu.sync_copy(hs_v, out.at[0, w])
        pltpu.sync_copy(hd_v, out.at[1, w])

    return deg


def _deg_col(part):
    ones = jnp.ones((_NW, 1), jnp.float32)
    return lax.dot_general(part, ones, (((0,), (0,)), ((), ())),
                           preferred_element_type=jnp.float32)


@functools.cache
def _make_segsum(D, ch):
    nch = _EPW // ch
    zrows = min(ch, (64 * 512) // D)

    @functools.partial(
        pl.kernel,
        out_type=jax.ShapeDtypeStruct((2, N, D), jnp.float32),
        mesh=_sc_mesh(),
        scratch_types=[
            pltpu.VMEM((nch, ch), jnp.int32),
            pltpu.VMEM((nch, ch), jnp.int32),
            pltpu.VMEM((ch, D), jnp.float32),
            pltpu.VMEM((ch, D), jnp.float32),
            pltpu.SemaphoreType.DMA,
            pltpu.SemaphoreType.DMA,
            pltpu.SemaphoreType.DMA,
            pltpu.SemaphoreType.DMA,
        ],
    )
    def seg(tab_hbm, src_hbm, dst_hbm, out_hbm, si_v, di_v, rows0, rows1,
            gs0, gs1, ss0, ss1):
        c = lax.axis_index("c")
        s = lax.axis_index("s")
        w = c * _NS + s
        rows = (rows0, rows1)
        gsem = (gs0, gs1)
        ssem = (ss0, ss1)

        _fill_zero(rows0, zrows, D)

        pltpu.sync_copy(src_hbm.at[pl.ds(w * nch, nch)], si_v)
        pltpu.sync_copy(dst_hbm.at[pl.ds(w * nch, nch)], di_v)

        def zcp(i, carry):
            pltpu.sync_copy(
                rows0.at[pl.ds(0, zrows)],
                out_hbm.at[c, pl.ds(s * _STR + i * zrows, zrows)])
            return carry
        lax.fori_loop(0, _STR // zrows, zcp, 0)
        plsc.subcore_barrier()

        gh = [None, None]
        sh = [None, None]
        gh[0] = pltpu.async_copy(tab_hbm.at[si_v.at[0]], rows[0], gsem[0])
        for i in range(nch):
            b = i % 2
            nb = (i + 1) % 2
            if i + 1 < nch:
                if sh[nb] is not None:
                    sh[nb].wait()
                gh[nb] = pltpu.async_copy(
                    tab_hbm.at[si_v.at[i + 1]], rows[nb], gsem[nb])
            gh[b].wait()
            sh[b] = pltpu.async_copy(
                rows[b], out_hbm.at[c].at[di_v.at[i]], ssem[b], add=True)
        for b in range(2):
            if sh[b] is not None:
                sh[b].wait()

    return seg


def _tc1_body(feat_ref, hs_ref, w_ref, out_ref):
    so = lax.rsqrt(jnp.maximum(_deg_col(hs_ref[0]), 1.0))
    out_ref[...] = jnp.dot(feat_ref[...] * so, w_ref[...],
                           preferred_element_type=jnp.float32)


_tc1 = pl.pallas_call(
    _tc1_body,
    out_shape=jax.ShapeDtypeStruct((N, 2 * HID_DIM), jnp.float32),
)


def _tc2_body(agg_ref, hs_ref, b1_ref, w2_ref, out_ref):
    so = lax.rsqrt(jnp.maximum(_deg_col(hs_ref[0]), 1.0))
    si = lax.rsqrt(jnp.maximum(_deg_col(hs_ref[1]), 1.0))
    h = agg_ref[0] + agg_ref[1]
    h = jnp.maximum(h * si + b1_ref[...], 0.0) * so
    out_ref[...] = jnp.dot(h, w2_ref[...], preferred_element_type=jnp.float32)


_tc2 = pl.pallas_call(
    _tc2_body,
    out_shape=jax.ShapeDtypeStruct((N, HID_DIM), jnp.float32),
)


def _tc3_body(agg_ref, hd_ref, b2_ref, fc1w_ref, fc1b_ref, fc2w_ref,
              fc2b_ref, kh_ref, out_ref, zn_s, k0_s, k1_s):
    i = pl.program_id(0)

    @pl.when(i == 0)
    def _():
        si = lax.rsqrt(jnp.maximum(_deg_col(hd_ref[1]), 1.0))
        h = agg_ref[0] + agg_ref[1]
        h = jnp.maximum(h * si + b2_ref[...], 0.0)
        p = jnp.dot(h, fc1w_ref[...], preferred_element_type=jnp.float32)
        p = p + fc1b_ref[...]
        p = jnp.where(p > 0.0, p, jnp.exp(jnp.minimum(p, 0.0)) - 1.0)
        z = jnp.dot(p, fc2w_ref[...], preferred_element_type=jnp.float32)
        z = z + fc2b_ref[...]
        nrm = jnp.sqrt(jnp.sum(z * z, axis=1, keepdims=True))
        zn_s[...] = z / jnp.maximum(nrm, 1e-12)
        k0_s[...] = jnp.zeros((1, N), jnp.float32)
        k1_s[...] = jnp.zeros((1, N), jnp.float32)

    zb = zn_s[pl.ds(i * _RBS, _RBS), :]
    sim = lax.dot_general(zb, zn_s[...], (((1,), (1,)), ((), ())),
                          preferred_element_type=jnp.float32)
    sim = jnp.exp(sim * (1.0 / TEMP))
    k0_s[...] += jnp.sum(sim * kh_ref[0], axis=0)[None, :]
    k1_s[...] += jnp.sum(sim * kh_ref[1], axis=0)[None, :]

    @pl.when(i == pl.num_programs(0) - 1)
    def _():
        k0 = k0_s[...]
        k1 = k1_s[...]
        mask = (k0 != 0.0) & (k1 != 0.0)
        denom = jnp.where(mask, k0 + k1, 1.0)
        ratio = jnp.where(mask, k0 / denom, 1.0)
        valid = mask & (ratio < ALPHA)
        loss = -jnp.sum(jnp.where(valid, jnp.log(jnp.where(valid, ratio, 1.0)),
                                  0.0), keepdims=True)
        out_ref[...] = loss


_tc3 = pl.pallas_call(
    _tc3_body,
    grid=(N // _RBS,),
    in_specs=[
        pl.BlockSpec((2, N, HID_DIM), lambda i: (0, 0, 0)),
        pl.BlockSpec((2, _NW, N), lambda i: (0, 0, 0)),
        pl.BlockSpec((1, HID_DIM), lambda i: (0, 0)),
        pl.BlockSpec((HID_DIM, OUT_DIM), lambda i: (0, 0)),
        pl.BlockSpec((1, OUT_DIM), lambda i: (0, 0)),
        pl.BlockSpec((OUT_DIM, HID_DIM), lambda i: (0, 0)),
        pl.BlockSpec((1, HID_DIM), lambda i: (0, 0)),
        pl.BlockSpec((2, _RBS, N), lambda i: (0, i, 0)),
    ],
    out_specs=pl.BlockSpec((1, 1), lambda i: (0, 0)),
    out_shape=jax.ShapeDtypeStruct((1, 1), jnp.float32),
    scratch_shapes=[
        pltpu.VMEM((N, HID_DIM), jnp.float32),
        pltpu.VMEM((1, N), jnp.float32),
        pltpu.VMEM((1, N), jnp.float32),
    ],
)


def kernel(feat, edge_index, khops, W1, b1, W2, b2, fc1_w, fc1_b, fc2_w,
           fc2_b):
    src = edge_index[0]
    dst = edge_index[1]
    src128 = src.reshape(E // _CH, _CH)
    dst128 = dst.reshape(E // _CH, _CH)

    src64 = src.reshape(E // 64, 64)
    dst64 = dst.reshape(E // 64, 64)

    hist = _make_deg_kernel()(src128, dst128)
    t1 = _tc1(feat, hist, W1)
    agg1 = _make_segsum(2 * HID_DIM, 64)(t1, src64, dst64)
    t2 = _tc2(agg1, hist, b1.reshape(1, -1), W2)
    agg2 = _make_segsum(HID_DIM, _CH)(t2, src128, dst128)
    loss = _tc3(agg2, hist, b2.reshape(1, -1), fc1_w, fc1_b.reshape(1, -1),
                fc2_w, fc2_b.reshape(1, -1), khops)
    return loss[0, 0]


# speedup vs baseline: 5.1096x; 1.0035x over previous
import functools

import jax
import jax.numpy as jnp
from jax import lax
from jax.experimental import pallas as pl
from jax.experimental.pallas import tpu as pltpu
from jax.experimental.pallas import tpu_sc as plsc

N = 4096
E = 65536
IN_DIM = 512
HID_DIM = 256
OUT_DIM = 128
TEMP = 0.5
ALPHA = 0.8

_NS = 16
_NW = 2 * _NS
_EPW = E // _NW
_CH = 128
_NCH = _EPW // _CH
_STR = N // _NS

_RBS = 256


def _sc_mesh():
    return plsc.VectorSubcoreMesh(core_axis_name="c", subcore_axis_name="s")


def _fill_zero(ref, rows, cols):
    def row(i, carry):
        def col(j, carry2):
            ref[i, pl.ds(j * 16, 16)] = jnp.zeros((16,), jnp.float32)
            return carry2
        return lax.fori_loop(0, cols // 16, col, carry)
    lax.fori_loop(0, rows, row, 0)


@functools.cache
def _make_deg_kernel():
    @functools.partial(
        pl.kernel,
        out_type=jax.ShapeDtypeStruct((2, _NW, N), jnp.float32),
        mesh=_sc_mesh(),
        scratch_types=[
            pltpu.VMEM((_NCH, _CH), jnp.int32),
            pltpu.VMEM((_NCH, _CH), jnp.int32),
            pltpu.VMEM((N,), jnp.float32),
            pltpu.VMEM((N,), jnp.float32),
        ],
        compiler_params=pltpu.CompilerParams(needs_layout_passes=False),
    )
    def deg(src_hbm, dst_hbm, out, si_v, di_v, hs_v, hd_v):
        c = lax.axis_index("c")
        s = lax.axis_index("s")
        w = c * _NS + s

        def z(i, carry):
            hs_v[pl.ds(i * 16, 16)] = jnp.zeros((16,), jnp.float32)
            hd_v[pl.ds(i * 16, 16)] = jnp.zeros((16,), jnp.float32)
            return carry
        lax.fori_loop(0, N // 16, z, 0)

        pltpu.sync_copy(src_hbm.at[pl.ds(w * _NCH, _NCH)], si_v)
        pltpu.sync_copy(dst_hbm.at[pl.ds(w * _NCH, _NCH)], di_v)

        ones16 = jnp.full((16,), 1.0, jnp.float32)

        def chunk(i, carry):
            def sub(j, carry2):
                plsc.addupdate_scatter(hs_v, [si_v[i, pl.ds(j * 16, 16)]],
                                       ones16)
                plsc.addupdate_scatter(hd_v, [di_v[i, pl.ds(j * 16, 16)]],
                                       ones16)
                return carry2
            return lax.fori_loop(0, _CH // 16, sub, carry)
        lax.fori_loop(0, _NCH, chunk, 0)

        pltpu.sync_copy(hs_v, out.at[0, w])
        pltpu.sync_copy(hd_v, out.at[1, w])

    return deg


def _deg_col(part):
    ones = jnp.ones((_NW, 1), jnp.float32)
    return lax.dot_general(part, ones, (((0,), (0,)), ((), ())),
                           preferred_element_type=jnp.float32)


@functools.cache
def _make_segsum(D, ch):
    nch = _EPW // ch
    zrows = min(ch, (64 * 512) // D)

    @functools.partial(
        pl.kernel,
        out_type=jax.ShapeDtypeStruct((2, N, D), jnp.float32),
        mesh=_sc_mesh(),
        scratch_types=[
            pltpu.VMEM((nch, ch), jnp.int32),
            pltpu.VMEM((nch, ch), jnp.int32),
            pltpu.VMEM((ch, D), jnp.float32),
            pltpu.VMEM((ch, D), jnp.float32),
            pltpu.SemaphoreType.DMA,
            pltpu.SemaphoreType.DMA,
            pltpu.SemaphoreType.DMA,
            pltpu.SemaphoreType.DMA,
        ],
    )
    def seg(tab_hbm, src_hbm, dst_hbm, out_hbm, si_v, di_v, rows0, rows1,
            gs0, gs1, ss0, ss1):
        c = lax.axis_index("c")
        s = lax.axis_index("s")
        w = c * _NS + s
        rows = (rows0, rows1)
        gsem = (gs0, gs1)
        ssem = (ss0, ss1)

        _fill_zero(rows0, zrows, D)

        pltpu.sync_copy(src_hbm.at[pl.ds(w * nch, nch)], si_v)
        pltpu.sync_copy(dst_hbm.at[pl.ds(w * nch, nch)], di_v)

        def zcp(i, carry):
            pltpu.sync_copy(
                rows0.at[pl.ds(0, zrows)],
                out_hbm.at[c, pl.ds(s * _STR + i * zrows, zrows)])
            return carry
        lax.fori_loop(0, _STR // zrows, zcp, 0)
        plsc.subcore_barrier()

        gh = [None, None]
        sh = [None, None]
        gh[0] = pltpu.async_copy(tab_hbm.at[si_v.at[0]], rows[0], gsem[0])
        for i in range(nch):
            b = i % 2
            nb = (i + 1) % 2
            if i + 1 < nch:
                if sh[nb] is not None:
                    sh[nb].wait()
                gh[nb] = pltpu.async_copy(
                    tab_hbm.at[si_v.at[i + 1]], rows[nb], gsem[nb])
            gh[b].wait()
            sh[b] = pltpu.async_copy(
                rows[b], out_hbm.at[c].at[di_v.at[i]], ssem[b], add=True)
        for b in range(2):
            if sh[b] is not None:
                sh[b].wait()

    return seg


def _tc1_body(feat_ref, hs_ref, w_ref, out_ref):
    so = lax.rsqrt(jnp.maximum(_deg_col(hs_ref[0]), 1.0))
    out_ref[...] = jnp.dot(feat_ref[...] * so, w_ref[...],
                           preferred_element_type=jnp.float32)


_tc1 = pl.pallas_call(
    _tc1_body,
    out_shape=jax.ShapeDtypeStruct((N, 2 * HID_DIM), jnp.float32),
)


def _tc2_body(agg_ref, hs_ref, b1_ref, w2_ref, out_ref):
    so = lax.rsqrt(jnp.maximum(_deg_col(hs_ref[0]), 1.0))
    si = lax.rsqrt(jnp.maximum(_deg_col(hs_ref[1]), 1.0))
    h = agg_ref[0] + agg_ref[1]
    h = jnp.maximum(h * si + b1_ref[...], 0.0) * so
    out_ref[...] = jnp.dot(h, w2_ref[...], preferred_element_type=jnp.float32)


_tc2 = pl.pallas_call(
    _tc2_body,
    out_shape=jax.ShapeDtypeStruct((N, HID_DIM), jnp.float32),
)


def _tc3_body(agg_ref, hd_ref, b2_ref, fc1w_ref, fc1b_ref, fc2w_ref,
              fc2b_ref, kh_ref, out_ref, zn_s, k0_s, k1_s):
    i = pl.program_id(0)

    @pl.when(i == 0)
    def _():
        si = lax.rsqrt(jnp.maximum(_deg_col(hd_ref[1]), 1.0))
        h = agg_ref[0] + agg_ref[1]
        h = jnp.maximum(h * si + b2_ref[...], 0.0)
        p = jnp.dot(h, fc1w_ref[...], preferred_element_type=jnp.float32)
        p = p + fc1b_ref[...]
        p = jnp.where(p > 0.0, p, jnp.exp(jnp.minimum(p, 0.0)) - 1.0)
        z = jnp.dot(p, fc2w_ref[...], preferred_element_type=jnp.float32)
        z = z + fc2b_ref[...]
        nrm = jnp.sqrt(jnp.sum(z * z, axis=1, keepdims=True))
        zn_s[...] = z / jnp.maximum(nrm, 1e-12)
        k0_s[...] = jnp.zeros((1, N), jnp.float32)
        k1_s[...] = jnp.zeros((1, N), jnp.float32)

    zb = zn_s[pl.ds(i * _RBS, _RBS), :] * (1.0 / TEMP)
    sim = lax.dot_general(zb, zn_s[...], (((1,), (1,)), ((), ())),
                          preferred_element_type=jnp.float32)
    sim = jnp.exp(sim)
    k0_s[...] += jnp.sum(sim * kh_ref[0], axis=0)[None, :]
    k1_s[...] += jnp.sum(sim * kh_ref[1], axis=0)[None, :]

    @pl.when(i == pl.num_programs(0) - 1)
    def _():
        k0 = k0_s[...]
        k1 = k1_s[...]
        mask = (k0 != 0.0) & (k1 != 0.0)
        denom = jnp.where(mask, k0 + k1, 1.0)
        ratio = jnp.where(mask, k0 / denom, 1.0)
        valid = mask & (ratio < ALPHA)
        loss = -jnp.sum(jnp.where(valid, jnp.log(jnp.where(valid, ratio, 1.0)),
                                  0.0), keepdims=True)
        out_ref[...] = loss


_tc3 = pl.pallas_call(
    _tc3_body,
    grid=(N // _RBS,),
    in_specs=[
        pl.BlockSpec((2, N, HID_DIM), lambda i: (0, 0, 0)),
        pl.BlockSpec((2, _NW, N), lambda i: (0, 0, 0)),
        pl.BlockSpec((1, HID_DIM), lambda i: (0, 0)),
        pl.BlockSpec((HID_DIM, OUT_DIM), lambda i: (0, 0)),
        pl.BlockSpec((1, OUT_DIM), lambda i: (0, 0)),
        pl.BlockSpec((OUT_DIM, HID_DIM), lambda i: (0, 0)),
        pl.BlockSpec((1, HID_DIM), lambda i: (0, 0)),
        pl.BlockSpec((2, _RBS, N), lambda i: (0, i, 0)),
    ],
    out_specs=pl.BlockSpec((1, 1), lambda i: (0, 0)),
    out_shape=jax.ShapeDtypeStruct((1, 1), jnp.float32),
    scratch_shapes=[
        pltpu.VMEM((N, HID_DIM), jnp.float32),
        pltpu.VMEM((1, N), jnp.float32),
        pltpu.VMEM((1, N), jnp.float32),
    ],
)


def kernel(feat, edge_index, khops, W1, b1, W2, b2, fc1_w, fc1_b, fc2_w,
           fc2_b):
    src = edge_index[0]
    dst = edge_index[1]
    src128 = src.reshape(E // _CH, _CH)
    dst128 = dst.reshape(E // _CH, _CH)

    src64 = src.reshape(E // 64, 64)
    dst64 = dst.reshape(E // 64, 64)

    hist = _make_deg_kernel()(src128, dst128)
    t1 = _tc1(feat, hist, W1)
    agg1 = _make_segsum(2 * HID_DIM, 64)(t1, src64, dst64)
    t2 = _tc2(agg1, hist, b1.reshape(1, -1), W2)
    agg2 = _make_segsum(HID_DIM, _CH)(t2, src128, dst128)
    loss = _tc3(agg2, hist, b2.reshape(1, -1), fc1_w, fc1_b.reshape(1, -1),
                fc2_w, fc2_b.reshape(1, -1), khops)
    return loss[0, 0]
